# baseline (speedup 1.0000x reference)
import functools

import jax
import jax.numpy as jnp
from jax import lax
from jax.experimental import pallas as pl
from jax.experimental.pallas import tpu as pltpu
from jax.experimental.pallas import tpu_sc as plsc

B = 16384
D = 32
NC = 2
NS = 16
L = 16
NW = NC * NS
BPW = B // NW
C = 128
NCH = BPW // C

_mesh = plsc.VectorSubcoreMesh(core_axis_name="c", subcore_axis_name="s")


@functools.partial(
    pl.kernel,
    out_type=(jax.ShapeDtypeStruct((B,), jnp.float32),
              jax.ShapeDtypeStruct((B,), jnp.float32)),
    mesh=_mesh,
    compiler_params=pltpu.CompilerParams(needs_layout_passes=False,
                                         use_tc_tiling_on_sc=False),
    scratch_types=(
        [pltpu.VMEM((C,), jnp.int32) for _ in range(5)]
        + [pltpu.VMEM((C, D), jnp.float32) for _ in range(10)]
        + [pltpu.VMEM((BPW,), jnp.float32) for _ in range(2)]
        + [pltpu.SemaphoreType.DMA]
    ),
)
def _complex_score_sc(h, t, nh, nt, r, re_ent, im_ent, re_rel, im_rel,
                      pos_out, neg_out,
                      hc, tc, nhc, ntc, rc,
                      reh_v, imh_v, ret_v, imt_v,
                      renh_v, imnh_v, rent_v, imnt_v,
                      rer_v, imr_v,
                      pos_v, neg_v, sem):
    wid = lax.axis_index("s") * NC + lax.axis_index("c")
    base = wid * BPW

    for c in range(NCH):
        cb = c * C
        idx_descs = [
            pltpu.async_copy(h.at[pl.ds(base + cb, C)], hc, sem),
            pltpu.async_copy(t.at[pl.ds(base + cb, C)], tc, sem),
            pltpu.async_copy(nh.at[pl.ds(base + cb, C)], nhc, sem),
            pltpu.async_copy(nt.at[pl.ds(base + cb, C)], ntc, sem),
            pltpu.async_copy(r.at[pl.ds(base + cb, C)], rc, sem),
        ]
        for dsc in idx_descs:
            dsc.wait()
        descs = [
            pltpu.async_copy(re_ent.at[hc], reh_v, sem),
            pltpu.async_copy(im_ent.at[hc], imh_v, sem),
            pltpu.async_copy(re_ent.at[tc], ret_v, sem),
            pltpu.async_copy(im_ent.at[tc], imt_v, sem),
            pltpu.async_copy(re_ent.at[nhc], renh_v, sem),
            pltpu.async_copy(im_ent.at[nhc], imnh_v, sem),
            pltpu.async_copy(re_ent.at[ntc], rent_v, sem),
            pltpu.async_copy(im_ent.at[ntc], imnt_v, sem),
            pltpu.async_copy(re_rel.at[rc], rer_v, sem),
            pltpu.async_copy(im_rel.at[rc], imr_v, sem),
        ]
        for dsc in descs:
            dsc.wait()

        def g_body(g, _, cb=cb):
            elem = lax.iota(jnp.int32, L) + g * L
            accp = jnp.zeros((L,), jnp.float32)
            accn = jnp.zeros((L,), jnp.float32)
            for d in range(D):
                idx = [elem, jnp.full((L,), d, jnp.int32)]
                reh = plsc.load_gather(reh_v, idx)
                imh = plsc.load_gather(imh_v, idx)
                ret = plsc.load_gather(ret_v, idx)
                imt = plsc.load_gather(imt_v, idx)
                renh = plsc.load_gather(renh_v, idx)
                imnh = plsc.load_gather(imnh_v, idx)
                rent = plsc.load_gather(rent_v, idx)
                imnt = plsc.load_gather(imnt_v, idx)
                rer = plsc.load_gather(rer_v, idx)
                imr = plsc.load_gather(imr_v, idx)
                accp = accp + rer * (reh * ret + imh * imt)
                accp = accp + imr * ((reh - imh) * imt)
                accn = accn + rer * (renh * rent + imnh * imnt)
                accn = accn + imr * ((renh - imnh) * imnt)
            pos_v[pl.ds(cb + g * L, L)] = accp
            neg_v[pl.ds(cb + g * L, L)] = accn
            return 0

        lax.fori_loop(0, C // L, g_body, 0)

    pltpu.sync_copy(pos_v, pos_out.at[pl.ds(base, BPW)])
    pltpu.sync_copy(neg_v, neg_out.at[pl.ds(base, BPW)])


def kernel(h, t, nh, nt, r, re_ent, im_ent, re_rel, im_rel):
    return _complex_score_sc(h.astype(jnp.int32), t.astype(jnp.int32),
                             nh.astype(jnp.int32), nt.astype(jnp.int32),
                             r.astype(jnp.int32),
                             re_ent, im_ent, re_rel, im_rel)
